# baseline (speedup 1.0000x reference)
import functools

import jax
import jax.numpy as jnp
from jax import lax
from jax.experimental import pallas as pl
from jax.experimental.pallas import tpu as pltpu
from jax.experimental.pallas import tpu_sc as plsc

NC = 2
NS = 16
NW = NC * NS
K = 96
TW = 128

_HI = lax.Precision.HIGHEST


def _make_edge_pass(n, nacc, e, hcpad, heads, c):
    ept = e // NW
    nchunks = ept // K
    rps = (nacc // NS) & ~7
    tail = nacc - rps * NS
    assert ept * NW == e and nchunks * K == ept and tail % 8 == 0

    npairs = (nchunks - 1) // 2
    assert npairs * 2 + 1 == nchunks

    mesh = plsc.VectorSubcoreMesh(core_axis_name="c", subcore_axis_name="s")

    idx_t = pltpu.VMEM((K,), jnp.int32)
    row_t = pltpu.VMEM((K, TW), jnp.float32)

    @functools.partial(
        pl.kernel,
        out_type=jax.ShapeDtypeStruct((NC, nacc, TW), jnp.float32),
        mesh=mesh,
        scratch_types=(
            [pltpu.VMEM((2 * K,), jnp.int32)] * 2 + [idx_t] * 2
            + [row_t] * 4
            + [pltpu.VMEM_SHARED((nacc, TW), jnp.float32)]
            + [pltpu.SemaphoreType.DMA] * 10
        ),
    )
    def edge_kernel(ei_hbm, dst_hbm, t_hbm, zeros_hbm, out_hbm,
                    sd0, sd1, ds0, ds1,
                    sr0, sr1, ob0, ob1, acc,
                    ig0, ig1, is0, is1, gs0, gs1, gd0, gd1, ss0, ss1):
        SDG, DIDXS = (sd0, sd1), (ds0, ds1)
        SROWS, OBUF = (sr0, sr1), (ob0, ob1)
        IG, IS, GS, GD, SS = (ig0, ig1), (is0, is1), (gs0, gs1), (gd0, gd1), \
            (ss0, ss1)

        cid = lax.axis_index("c")
        sid = lax.axis_index("s")
        wid = sid * NC + cid
        r0 = sid * rps
        base = wid * ept

        pltpu.sync_copy(zeros_hbm.at[pl.ds(r0, rps)], acc.at[pl.ds(r0, rps)])

        @pl.when(sid == 0)
        def _zero_tail():
            pltpu.sync_copy(zeros_hbm.at[pl.ds(rps * NS, tail)],
                            acc.at[pl.ds(rps * NS, tail)])

        plsc.subcore_barrier()

        def idxg_issue(ci, b):
            o = (wid * nchunks + ci) * 2 * K
            pltpu.async_copy(ei_hbm.at[pl.ds(o, 2 * K)], SDG[b], IG[b])

        def idxg_wait(b):
            pltpu.make_async_copy(ei_hbm.at[pl.ds(0, 2 * K)], SDG[b],
                                  IG[b]).wait()

        def idxs_issue(ci, b):
            pltpu.async_copy(dst_hbm.at[pl.ds(base + ci * K, K)], DIDXS[b],
                             IS[b])

        def idxs_wait(b):
            pltpu.make_async_copy(dst_hbm.at[pl.ds(0, K)], DIDXS[b],
                                  IS[b]).wait()

        def gsrc_issue(b):
            pltpu.async_copy(t_hbm.at[SDG[b].at[pl.ds(0, K)]], SROWS[b],
                             GS[b])

        def gsrc_wait(b):
            pltpu.make_async_copy(t_hbm.at[SDG[b].at[pl.ds(0, K)]], SROWS[b],
                                  GS[b]).wait()

        def gdst_issue(b):
            pltpu.async_copy(t_hbm.at[SDG[b].at[pl.ds(K, K)]], OBUF[b],
                             GD[b])

        def gdst_wait(b):
            pltpu.make_async_copy(t_hbm.at[SDG[b].at[pl.ds(K, K)]], OBUF[b],
                                  GD[b]).wait()

        def scatter_issue(b):
            pltpu.async_copy(OBUF[b], acc.at[DIDXS[b]], SS[b], add=True)

        def scatter_wait(b):
            pltpu.make_async_copy(OBUF[b], acc.at[DIDXS[b]], SS[b]).wait()

        def compute(b):
            srows, obuf = SROWS[b], OBUF[b]

            @pl.loop(0, K)
            def _edge(ei):
                a = srows[ei, pl.ds(hcpad, 16)]
                d = obuf[ei, pl.ds(hcpad + 16, 16)]
                t = a + d
                w16 = jnp.exp(jnp.maximum(t, 0.2 * t))
                obuf[ei, pl.ds(hcpad, 16)] = w16
                if heads == 1:
                    wv = jnp.full((16,), w16[0], jnp.float32)
                    obuf[ei, pl.ds(0, 16)] = srows[ei, pl.ds(0, 16)] * wv
                elif c == 16:
                    for q in range(hcpad // 16):
                        wv = jnp.full((16,), w16[q], jnp.float32)
                        obuf[ei, pl.ds(16 * q, 16)] = (
                            srows[ei, pl.ds(16 * q, 16)] * wv)
                else:
                    lo = lax.iota(jnp.int32, 16) < 8
                    for q in range(hcpad // 16):
                        wa = jnp.full((16,), w16[2 * q], jnp.float32)
                        wb = jnp.full((16,), w16[2 * q + 1], jnp.float32)
                        wv = jnp.where(lo, wa, wb)
                        obuf[ei, pl.ds(16 * q, 16)] = (
                            srows[ei, pl.ds(16 * q, 16)] * wv)

        def step(ci, b, j=None, first=False, last=False):
            nb = 1 - b
            if not last:
                idxg_wait(nb)
                gsrc_issue(nb)
            gsrc_wait(b)
            gdst_wait(b)
            idxs_wait(b)
            compute(b)
            scatter_issue(b)

            if first:
                pass
            elif b == 0 and j is not None:
                pl.when(j > 0)(lambda: scatter_wait(nb))
            else:
                scatter_wait(nb)

            if not last:
                gdst_issue(nb)
                idxs_issue(ci + 1, nb)
                if b == 0:
                    idxg_issue(ci + 2, b)
                else:
                    pl.when(j < npairs - 1)(lambda: idxg_issue(ci + 2, b))

        idxg_issue(0, 0)
        idxg_issue(1, 1)
        idxs_issue(0, 0)
        idxg_wait(0)
        gsrc_issue(0)
        gdst_issue(0)

        @pl.loop(0, npairs)
        def _pair(j):
            step(2 * j, 0, j=j, first=False)
            step(2 * j + 1, 1, j=j)

        step(nchunks - 1, 0, last=True)
        scatter_wait(0)

        plsc.subcore_barrier()
        pltpu.sync_copy(acc.at[pl.ds(r0, rps)], out_hbm.at[cid, pl.ds(r0, rps)])

        @pl.when(sid == 0)
        def _out_tail():
            pltpu.sync_copy(acc.at[pl.ds(rps * NS, tail)],
                            out_hbm.at[cid, pl.ds(rps * NS, tail)])

    return edge_kernel


def _write_table(t_ref, h, as16, ad16, hcpad):
    t_ref[:, 0:hcpad] = h
    t_ref[:, hcpad:hcpad + 16] = as16
    t_ref[:, hcpad + 16:hcpad + 32] = ad16
    if hcpad + 32 < TW:
        t_ref[:, hcpad + 32:TW] = jnp.zeros(
            (h.shape[0], TW - hcpad - 32), jnp.float32)


def _tc_prep0(x, w1, a1s, a1d, hcpad):
    n = x.shape[0]

    def body(x_ref, w_ref, as_a, ad_a, t_ref):
        h = jnp.dot(x_ref[...], w_ref[...], precision=_HI)
        as16 = jnp.dot(h, as_a[...], precision=_HI)
        ad16 = jnp.dot(h, ad_a[...], precision=_HI)
        _write_table(t_ref, h, as16, ad16, hcpad)

    return pl.pallas_call(
        body,
        out_shape=jax.ShapeDtypeStruct((n, TW), jnp.float32),
    )(x, w1, a1s, a1d)


def _merge(acc, t_prev, heads, c, hcpad):
    accs = acc[0] + acc[1]
    as_prev = t_prev[:, hcpad:hcpad + 16]
    ad_prev = t_prev[:, hcpad + 16:hcpad + 32]
    t = as_prev + ad_prev
    wself = jnp.exp(jnp.maximum(t, 0.2 * t))
    out = jnp.zeros_like(t_prev[:, 0:c])
    for j in range(heads):
        num = (accs[:, j * c:(j + 1) * c]
               + wself[:, j:j + 1] * t_prev[:, j * c:(j + 1) * c])
        den = accs[:, hcpad + j:hcpad + j + 1] + wself[:, j:j + 1]
        out = out + num / (den + 1e-16)
    return out * (1.0 / heads)


_BN = 2000


def _tc_merge_prep(acc, t_prev, b, wn, ans, andm, heads, c, hcpad, hcpad_next):
    n = t_prev.shape[0]
    grid = n // _BN

    def body(acc_ref, t_ref, b_ref, wn_ref, ans_ref, and_ref, tn_ref):
        out = _merge(acc_ref[...], t_ref[...], heads, c, hcpad)
        xnext = jnp.maximum(out + b_ref[...], 0.0)
        hn = jnp.dot(xnext, wn_ref[...], precision=_HI)
        asn = jnp.dot(hn, ans_ref[...], precision=_HI)
        adn = jnp.dot(hn, and_ref[...], precision=_HI)
        _write_table(tn_ref, hn, asn, adn, hcpad_next)

    return pl.pallas_call(
        body,
        grid=(grid,),
        in_specs=[
            pl.BlockSpec((2, _BN, TW), lambda i: (0, i, 0)),
            pl.BlockSpec((_BN, TW), lambda i: (i, 0)),
            pl.BlockSpec(b.shape, lambda i: (0, 0)),
            pl.BlockSpec(wn.shape, lambda i: (0, 0)),
            pl.BlockSpec(ans.shape, lambda i: (0, 0)),
            pl.BlockSpec(andm.shape, lambda i: (0, 0)),
        ],
        out_specs=pl.BlockSpec((_BN, TW), lambda i: (i, 0)),
        out_shape=jax.ShapeDtypeStruct((n, TW), jnp.float32),
    )(acc, t_prev, b, wn, ans, andm)


def _tc_final(acc, t_prev, b, heads, c, hcpad):
    n = t_prev.shape[0]
    grid = n // _BN

    def body(acc_ref, t_ref, b_ref, o_ref):
        out = _merge(acc_ref[...], t_ref[...], heads, c, hcpad)
        out = out + b_ref[...]
        m = jnp.max(out, axis=1, keepdims=True)
        z = out - m
        lse = jnp.log(jnp.sum(jnp.exp(z), axis=1, keepdims=True))
        o_ref[...] = z - lse

    return pl.pallas_call(
        body,
        grid=(grid,),
        in_specs=[
            pl.BlockSpec((2, _BN, TW), lambda i: (0, i, 0)),
            pl.BlockSpec((_BN, TW), lambda i: (i, 0)),
            pl.BlockSpec(b.shape, lambda i: (0, 0)),
        ],
        out_specs=pl.BlockSpec((_BN, c), lambda i: (i, 0)),
        out_shape=jax.ShapeDtypeStruct((n, c), jnp.float32),
    )(acc, t_prev, b)


def _amat(a, hcpad):
    heads, c = a.shape
    m = jnp.zeros((hcpad, 16), jnp.float32)
    for j in range(heads):
        m = m.at[j * c:(j + 1) * c, j].set(a[j])
    return m


def kernel(x, edge_index, W1, a_src1, a_dst1, b1, W2, a_src2, a_dst2, b2,
           W3, a_src3, a_dst3, b3, W4, a_src4, a_dst4, b4):
    n = x.shape[0]
    e = edge_index.shape[1]

    ept = e // NW
    ept_pad = ((ept + K - 1) // K) * K
    if ept_pad % (2 * K) == 0:
        ept_pad += K
    nacc = n + 8
    pad = ept_pad - ept
    src = jnp.pad(edge_index[0].reshape(NW, ept), ((0, 0), (0, pad)),
                  constant_values=0).reshape(-1)
    dst = jnp.pad(edge_index[1].reshape(NW, ept), ((0, 0), (0, pad)),
                  constant_values=n).reshape(-1)
    e_pad = NW * ept_pad
    nch = ept_pad // K
    ei = jnp.stack([src.reshape(NW, nch, K), dst.reshape(NW, nch, K)],
                   axis=2).reshape(-1)

    cfg1, cfg2, cfg3, cfg4 = (6, 8, 48), (6, 16, 96), (1, 8, 16), (1, 16, 16)

    a1s, a1d = _amat(a_src1, 48), _amat(a_dst1, 48)
    a2s, a2d = _amat(a_src2, 96), _amat(a_dst2, 96)
    a3s, a3d = _amat(a_src3, 16), _amat(a_dst3, 16)
    a4s, a4d = _amat(a_src4, 16), _amat(a_dst4, 16)
    w3p = jnp.zeros((16, 16), jnp.float32).at[:, :8].set(W3)

    ep1 = _make_edge_pass(n, nacc, e_pad, 48, 6, 8)
    ep2 = _make_edge_pass(n, nacc, e_pad, 96, 6, 16)
    ep3 = _make_edge_pass(n, nacc, e_pad, 16, 1, 8)
    ep4 = _make_edge_pass(n, nacc, e_pad, 16, 1, 16)

    z = jnp.zeros((nacc, TW), jnp.float32)
    _pad = lambda t: jnp.pad(t, ((0, nacc - n), (0, 0)))

    t1 = _tc_prep0(x, W1, a1s, a1d, 48)
    p1 = ep1(ei, dst, _pad(t1), z)[:, :n]
    t2 = _tc_merge_prep(p1, t1, b1.reshape(1, -1), W2, a2s, a2d, *cfg1, 96)
    p2 = ep2(ei, dst, _pad(t2), z)[:, :n]
    t3 = _tc_merge_prep(p2, t2, b2.reshape(1, -1), w3p, a3s, a3d, *cfg2, 16)
    p3 = ep3(ei, dst, _pad(t3), z)[:, :n]
    t4 = _tc_merge_prep(p3, t3, b3.reshape(1, -1), W4, a4s, a4d, *cfg3, 16)
    p4 = ep4(ei, dst, _pad(t4), z)[:, :n]
    return _tc_final(p4, t4, b4.reshape(1, -1), *cfg4)


# speedup vs baseline: 1.4567x; 1.4567x over previous
import functools

import jax
import jax.numpy as jnp
from jax import lax
from jax.experimental import pallas as pl
from jax.experimental.pallas import tpu as pltpu
from jax.experimental.pallas import tpu_sc as plsc

NC = 2
NS = 16
NW = NC * NS
K = 80
TW = 128

_HI = lax.Precision.HIGHEST


def _make_edge_pass(n, nacc, e, hcpad, heads, c):
    ept = e // NW
    nchunks = ept // K
    rps = (nacc // NS) & ~7
    tail = nacc - rps * NS
    assert ept * NW == e and nchunks * K == ept and tail % 8 == 0

    npairs = (nchunks - 1) // 2
    assert npairs * 2 + 1 == nchunks

    mesh = plsc.VectorSubcoreMesh(core_axis_name="c", subcore_axis_name="s")

    idx_t = pltpu.VMEM((K,), jnp.int32)
    row_t = pltpu.VMEM((K, TW), jnp.float32)

    @functools.partial(
        pl.kernel,
        out_type=jax.ShapeDtypeStruct((NC, nacc, TW), jnp.float32),
        mesh=mesh,
        scratch_types=(
            [idx_t] * 6 + [row_t] * 4
            + [pltpu.VMEM_SHARED((nacc, TW), jnp.float32)]
            + [pltpu.SemaphoreType.DMA] * 10
        ),
    )
    def edge_kernel(src_hbm, dst_hbm, t_hbm, zeros_hbm, out_hbm,
                    s0, s1, dg0, dg1, ds0, ds1,
                    sr0, sr1, ob0, ob1, acc,
                    ig0, ig1, is0, is1, gs0, gs1, gd0, gd1, ss0, ss1):
        SIDX, DIDXG, DIDXS = (s0, s1), (dg0, dg1), (ds0, ds1)
        SROWS, OBUF = (sr0, sr1), (ob0, ob1)
        IG, IS, GS, GD, SS = (ig0, ig1), (is0, is1), (gs0, gs1), (gd0, gd1), \
            (ss0, ss1)

        cid = lax.axis_index("c")
        sid = lax.axis_index("s")
        wid = sid * NC + cid
        r0 = sid * rps
        base = wid * ept

        pltpu.sync_copy(zeros_hbm.at[pl.ds(r0, rps)], acc.at[pl.ds(r0, rps)])

        @pl.when(sid == 0)
        def _zero_tail():
            pltpu.sync_copy(zeros_hbm.at[pl.ds(rps * NS, tail)],
                            acc.at[pl.ds(rps * NS, tail)])

        plsc.subcore_barrier()

        def idxg_issue(ci, b):
            o = base + ci * K
            pltpu.async_copy(src_hbm.at[pl.ds(o, K)], SIDX[b], IG[b])
            pltpu.async_copy(dst_hbm.at[pl.ds(o, K)], DIDXG[b], IG[b])

        def idxg_wait(b):
            pltpu.make_async_copy(src_hbm.at[pl.ds(0, K)], SIDX[b],
                                  IG[b]).wait()
            pltpu.make_async_copy(dst_hbm.at[pl.ds(0, K)], DIDXG[b],
                                  IG[b]).wait()

        def idxs_issue(ci, b):
            pltpu.async_copy(dst_hbm.at[pl.ds(base + ci * K, K)], DIDXS[b],
                             IS[b])

        def idxs_wait(b):
            pltpu.make_async_copy(dst_hbm.at[pl.ds(0, K)], DIDXS[b],
                                  IS[b]).wait()

        def gsrc_issue(b):
            pltpu.async_copy(t_hbm.at[SIDX[b]], SROWS[b], GS[b])

        def gsrc_wait(b):
            pltpu.make_async_copy(t_hbm.at[SIDX[b]], SROWS[b], GS[b]).wait()

        def gdst_issue(b):
            pltpu.async_copy(t_hbm.at[DIDXG[b]], OBUF[b], GD[b])

        def gdst_wait(b):
            pltpu.make_async_copy(t_hbm.at[DIDXG[b]], OBUF[b], GD[b]).wait()

        def scatter_issue(b):
            pltpu.async_copy(OBUF[b], acc.at[DIDXS[b]], SS[b], add=True)

        def scatter_wait(b):
            pltpu.make_async_copy(OBUF[b], acc.at[DIDXS[b]], SS[b]).wait()

        def compute(b):
            srows, obuf = SROWS[b], OBUF[b]

            @pl.loop(0, K)
            def _edge(ei):
                a = srows[ei, pl.ds(hcpad, 16)]
                d = obuf[ei, pl.ds(hcpad + 16, 16)]
                t = a + d
                w16 = jnp.exp(jnp.maximum(t, 0.2 * t))
                obuf[ei, pl.ds(hcpad, 16)] = w16
                if heads == 1:
                    wv = jnp.full((16,), w16[0], jnp.float32)
                    obuf[ei, pl.ds(0, 16)] = srows[ei, pl.ds(0, 16)] * wv
                elif c == 16:
                    for q in range(hcpad // 16):
                        wv = jnp.full((16,), w16[q], jnp.float32)
                        obuf[ei, pl.ds(16 * q, 16)] = (
                            srows[ei, pl.ds(16 * q, 16)] * wv)
                else:
                    lo = lax.iota(jnp.int32, 16) < 8
                    for q in range(hcpad // 16):
                        wa = jnp.full((16,), w16[2 * q], jnp.float32)
                        wb = jnp.full((16,), w16[2 * q + 1], jnp.float32)
                        wv = jnp.where(lo, wa, wb)
                        obuf[ei, pl.ds(16 * q, 16)] = (
                            srows[ei, pl.ds(16 * q, 16)] * wv)

        def step(ci, b, j=None, first=False, last=False):
            nb = 1 - b
            if not last:
                idxg_wait(nb)
                gsrc_issue(nb)
            gsrc_wait(b)
            gdst_wait(b)
            idxs_wait(b)
            compute(b)
            scatter_issue(b)

            if first:
                pass
            elif b == 0 and j is not None:
                pl.when(j > 0)(lambda: scatter_wait(nb))
            else:
                scatter_wait(nb)

            if not last:
                gdst_issue(nb)
                idxs_issue(ci + 1, nb)
                if b == 0:
                    idxg_issue(ci + 2, b)
                else:
                    pl.when(j < npairs - 1)(lambda: idxg_issue(ci + 2, b))

        idxg_issue(0, 0)
        idxg_issue(1, 1)
        idxs_issue(0, 0)
        idxg_wait(0)
        gsrc_issue(0)
        gdst_issue(0)

        @pl.loop(0, npairs)
        def _pair(j):
            step(2 * j, 0, j=j, first=False)
            step(2 * j + 1, 1, j=j)

        step(nchunks - 1, 0, last=True)
        scatter_wait(0)

        plsc.subcore_barrier()
        pltpu.sync_copy(acc.at[pl.ds(r0, rps)], out_hbm.at[cid, pl.ds(r0, rps)])

        @pl.when(sid == 0)
        def _out_tail():
            pltpu.sync_copy(acc.at[pl.ds(rps * NS, tail)],
                            out_hbm.at[cid, pl.ds(rps * NS, tail)])

    return edge_kernel


def _write_table(t_ref, h, as16, ad16, hcpad):
    t_ref[:, 0:hcpad] = h
    t_ref[:, hcpad:hcpad + 16] = as16
    t_ref[:, hcpad + 16:hcpad + 32] = ad16
    if hcpad + 32 < TW:
        t_ref[:, hcpad + 32:TW] = jnp.zeros(
            (h.shape[0], TW - hcpad - 32), jnp.float32)


def _tc_prep0(x, w1, a1s, a1d, hcpad):
    n = x.shape[0]

    def body(x_ref, w_ref, as_a, ad_a, t_ref):
        h = jnp.dot(x_ref[...], w_ref[...], precision=_HI)
        as16 = jnp.dot(h, as_a[...], precision=_HI)
        ad16 = jnp.dot(h, ad_a[...], precision=_HI)
        _write_table(t_ref, h, as16, ad16, hcpad)

    return pl.pallas_call(
        body,
        out_shape=jax.ShapeDtypeStruct((n, TW), jnp.float32),
    )(x, w1, a1s, a1d)


def _merge(acc, t_prev, heads, c, hcpad):
    accs = acc[0] + acc[1]
    as_prev = t_prev[:, hcpad:hcpad + 16]
    ad_prev = t_prev[:, hcpad + 16:hcpad + 32]
    t = as_prev + ad_prev
    wself = jnp.exp(jnp.maximum(t, 0.2 * t))
    out = jnp.zeros_like(t_prev[:, 0:c])
    for j in range(heads):
        num = (accs[:, j * c:(j + 1) * c]
               + wself[:, j:j + 1] * t_prev[:, j * c:(j + 1) * c])
        den = accs[:, hcpad + j:hcpad + j + 1] + wself[:, j:j + 1]
        out = out + num / (den + 1e-16)
    return out * (1.0 / heads)


_BN = 2000


def _tc_merge_prep(acc, t_prev, b, wn, ans, andm, heads, c, hcpad, hcpad_next):
    n = t_prev.shape[0]
    grid = n // _BN

    def body(acc_ref, t_ref, b_ref, wn_ref, ans_ref, and_ref, tn_ref):
        out = _merge(acc_ref[...], t_ref[...], heads, c, hcpad)
        xnext = jnp.maximum(out + b_ref[...], 0.0)
        hn = jnp.dot(xnext, wn_ref[...], precision=_HI)
        asn = jnp.dot(hn, ans_ref[...], precision=_HI)
        adn = jnp.dot(hn, and_ref[...], precision=_HI)
        _write_table(tn_ref, hn, asn, adn, hcpad_next)

    return pl.pallas_call(
        body,
        grid=(grid,),
        in_specs=[
            pl.BlockSpec((2, _BN, TW), lambda i: (0, i, 0)),
            pl.BlockSpec((_BN, TW), lambda i: (i, 0)),
            pl.BlockSpec(b.shape, lambda i: (0, 0)),
            pl.BlockSpec(wn.shape, lambda i: (0, 0)),
            pl.BlockSpec(ans.shape, lambda i: (0, 0)),
            pl.BlockSpec(andm.shape, lambda i: (0, 0)),
        ],
        out_specs=pl.BlockSpec((_BN, TW), lambda i: (i, 0)),
        out_shape=jax.ShapeDtypeStruct((n, TW), jnp.float32),
    )(acc, t_prev, b, wn, ans, andm)


def _tc_final(acc, t_prev, b, heads, c, hcpad):
    n = t_prev.shape[0]
    grid = n // _BN

    def body(acc_ref, t_ref, b_ref, o_ref):
        out = _merge(acc_ref[...], t_ref[...], heads, c, hcpad)
        out = out + b_ref[...]
        m = jnp.max(out, axis=1, keepdims=True)
        z = out - m
        lse = jnp.log(jnp.sum(jnp.exp(z), axis=1, keepdims=True))
        o_ref[...] = z - lse

    return pl.pallas_call(
        body,
        grid=(grid,),
        in_specs=[
            pl.BlockSpec((2, _BN, TW), lambda i: (0, i, 0)),
            pl.BlockSpec((_BN, TW), lambda i: (i, 0)),
            pl.BlockSpec(b.shape, lambda i: (0, 0)),
        ],
        out_specs=pl.BlockSpec((_BN, c), lambda i: (i, 0)),
        out_shape=jax.ShapeDtypeStruct((n, c), jnp.float32),
    )(acc, t_prev, b)


def _amat(a, hcpad):
    heads, c = a.shape
    m = jnp.zeros((hcpad, 16), jnp.float32)
    for j in range(heads):
        m = m.at[j * c:(j + 1) * c, j].set(a[j])
    return m


def kernel(x, edge_index, W1, a_src1, a_dst1, b1, W2, a_src2, a_dst2, b2,
           W3, a_src3, a_dst3, b3, W4, a_src4, a_dst4, b4):
    n = x.shape[0]
    e = edge_index.shape[1]

    ept = e // NW
    ept_pad = ((ept + K - 1) // K) * K
    if ept_pad % (2 * K) == 0:
        ept_pad += K
    pad = ept_pad - ept
    nacc = n + 8 if pad else n
    if pad:
        src = jnp.pad(edge_index[0].reshape(NW, ept), ((0, 0), (0, pad)),
                      constant_values=0).reshape(-1)
        dst = jnp.pad(edge_index[1].reshape(NW, ept), ((0, 0), (0, pad)),
                      constant_values=n).reshape(-1)
    else:
        src, dst = edge_index[0], edge_index[1]
    e_pad = NW * ept_pad

    cfg1, cfg2, cfg3, cfg4 = (6, 8, 48), (6, 16, 96), (1, 8, 16), (1, 16, 16)

    a1s, a1d = _amat(a_src1, 48), _amat(a_dst1, 48)
    a2s, a2d = _amat(a_src2, 96), _amat(a_dst2, 96)
    a3s, a3d = _amat(a_src3, 16), _amat(a_dst3, 16)
    a4s, a4d = _amat(a_src4, 16), _amat(a_dst4, 16)
    w3p = jnp.zeros((16, 16), jnp.float32).at[:, :8].set(W3)

    ep1 = _make_edge_pass(n, nacc, e_pad, 48, 6, 8)
    ep2 = _make_edge_pass(n, nacc, e_pad, 96, 6, 16)
    ep3 = _make_edge_pass(n, nacc, e_pad, 16, 1, 8)
    ep4 = _make_edge_pass(n, nacc, e_pad, 16, 1, 16)

    z = jnp.zeros((nacc, TW), jnp.float32)
    if pad:
        _pad = lambda t: jnp.pad(t, ((0, nacc - n), (0, 0)))
    else:
        _pad = lambda t: t

    t1 = _tc_prep0(x, W1, a1s, a1d, 48)
    p1 = ep1(src, dst, _pad(t1), z)[:, :n]
    t2 = _tc_merge_prep(p1, t1, b1.reshape(1, -1), W2, a2s, a2d, *cfg1, 96)
    p2 = ep2(src, dst, _pad(t2), z)[:, :n]
    t3 = _tc_merge_prep(p2, t2, b2.reshape(1, -1), w3p, a3s, a3d, *cfg2, 16)
    p3 = ep3(src, dst, _pad(t3), z)[:, :n]
    t4 = _tc_merge_prep(p3, t3, b3.reshape(1, -1), W4, a4s, a4d, *cfg3, 16)
    p4 = ep4(src, dst, _pad(t4), z)[:, :n]
    return _tc_final(p4, t4, b4.reshape(1, -1), *cfg4)
